# Initial kernel scaffold; baseline (speedup 1.0000x reference)
import functools

import jax
import jax.numpy as jnp
from jax import lax
from jax.experimental import pallas as pl
from jax.experimental.pallas import tpu as pltpu
from jax.experimental.pallas import tpu_sc as plsc

N_NODES = 50000
N_EDGES = 800000
IN_FEATS = 16
OUT_FEATS = 100

NUM_CORES = 2
NUM_SUBCORES = 16
NUM_WORKERS = NUM_CORES * NUM_SUBCORES
GROUP = 128
NUM_GROUPS = N_EDGES // GROUP
ROWS_PER_TILE = N_NODES // NUM_SUBCORES
ZROWS = 625


def _sc_segment_sum(e, dst):
  mesh = plsc.VectorSubcoreMesh(core_axis_name="c", subcore_axis_name="s")

  @functools.partial(
      pl.kernel,
      mesh=mesh,
      out_type=[
          jax.ShapeDtypeStruct((NUM_CORES, N_NODES, IN_FEATS), jnp.float32),
          jax.ShapeDtypeStruct((NUM_CORES, N_NODES, IN_FEATS), jnp.float32),
      ],
      scratch_types=[
          pltpu.VMEM((GROUP, IN_FEATS), jnp.float32),
          pltpu.VMEM((1, GROUP), jnp.int32),
          pltpu.VMEM((GROUP, IN_FEATS), jnp.float32),
          pltpu.VMEM((ZROWS, IN_FEATS), jnp.float32),
          pltpu.VMEM_SHARED((N_NODES, IN_FEATS), jnp.float32),
          pltpu.VMEM_SHARED((N_NODES, IN_FEATS), jnp.float32),
      ],
  )
  def sc_kernel(e_hbm, dst_hbm, seg_out, cnt_out,
                rows_v, idx_v, ones_v, zero_v, seg_sh, cnt_sh):
    c = lax.axis_index("c")
    s = lax.axis_index("s")
    wid = c * NUM_SUBCORES + s

    def init_body(i, _):
      zero_v[i] = jnp.zeros((IN_FEATS,), jnp.float32)
      return 0

    lax.fori_loop(0, ZROWS, init_body, 0)

    def ones_body(i, _):
      ones_v[i] = jnp.ones((IN_FEATS,), jnp.float32)
      return 0

    lax.fori_loop(0, GROUP, ones_body, 0)

    row0 = s * ROWS_PER_TILE
    for k in range(ROWS_PER_TILE // ZROWS):
      pltpu.sync_copy(zero_v, seg_sh.at[pl.ds(row0 + k * ZROWS, ZROWS)])
      pltpu.sync_copy(zero_v, cnt_sh.at[pl.ds(row0 + k * ZROWS, ZROWS)])
    plsc.subcore_barrier()

    g0 = wid * NUM_GROUPS // NUM_WORKERS
    g1 = (wid + 1) * NUM_GROUPS // NUM_WORKERS

    def group_body(g, _):
      base = g * GROUP
      pltpu.sync_copy(dst_hbm.at[pl.ds(base, GROUP)], idx_v.at[0])
      pltpu.sync_copy(e_hbm.at[pl.ds(base, GROUP)], rows_v)
      pltpu.sync_copy(rows_v, seg_sh.at[idx_v.at[0]], add=True)
      pltpu.sync_copy(ones_v, cnt_sh.at[idx_v.at[0]], add=True)
      return 0

    lax.fori_loop(g0, g1, group_body, 0)
    plsc.subcore_barrier()

    pltpu.sync_copy(seg_sh.at[pl.ds(row0, ROWS_PER_TILE)],
                    seg_out.at[c, pl.ds(row0, ROWS_PER_TILE)])
    pltpu.sync_copy(cnt_sh.at[pl.ds(row0, ROWS_PER_TILE)],
                    cnt_out.at[c, pl.ds(row0, ROWS_PER_TILE)])

  return sc_kernel(e, dst)


ROW_BLOCK = 1250


def _tc_body(seg_ref, cnt_ref, w_ref, b_ref, out_ref):
  seg = seg_ref[0] + seg_ref[1]
  cnt = cnt_ref[0, :, 0:1] + cnt_ref[1, :, 0:1]
  acc = jnp.dot(seg, w_ref[...], preferred_element_type=jnp.float32)
  acc = acc + cnt * b_ref[...]
  out_ref[...] = acc / jnp.maximum(cnt, 1.0)


def _tc_finish(seg, cnt, W, b2d):
  grid = N_NODES // ROW_BLOCK
  return pl.pallas_call(
      _tc_body,
      grid=(grid,),
      in_specs=[
          pl.BlockSpec((NUM_CORES, ROW_BLOCK, IN_FEATS), lambda i: (0, i, 0)),
          pl.BlockSpec((NUM_CORES, ROW_BLOCK, IN_FEATS), lambda i: (0, i, 0)),
          pl.BlockSpec((IN_FEATS, OUT_FEATS), lambda i: (0, 0)),
          pl.BlockSpec((1, OUT_FEATS), lambda i: (0, 0)),
      ],
      out_specs=pl.BlockSpec((ROW_BLOCK, OUT_FEATS), lambda i: (i, 0)),
      out_shape=jax.ShapeDtypeStruct((N_NODES, OUT_FEATS), jnp.float32),
  )(seg, cnt, W, b2d)


@jax.jit
def kernel(e, edge_index, W, b):
  dst = edge_index[1]
  seg, cnt = _sc_segment_sum(e, dst)
  return _tc_finish(seg, cnt, W, b.reshape(1, OUT_FEATS))


# speedup vs baseline: 4.3498x; 4.3498x over previous
import functools

import jax
import jax.numpy as jnp
from jax import lax
from jax.experimental import pallas as pl
from jax.experimental.pallas import tpu as pltpu
from jax.experimental.pallas import tpu_sc as plsc

N_NODES = 50000
N_EDGES = 800000
IN_FEATS = 16
OUT_FEATS = 100

NUM_CORES = 2
NUM_SUBCORES = 16
NUM_WORKERS = NUM_CORES * NUM_SUBCORES
GROUP = 128
NUM_GROUPS = N_EDGES // GROUP
NPAD = 50048
ROWS_PER_TILE = NPAD // NUM_SUBCORES
ZROWS = 1000


def _sc_segment_sum(e, dst):
  mesh = plsc.VectorSubcoreMesh(core_axis_name="c", subcore_axis_name="s")

  @functools.partial(
      pl.kernel,
      mesh=mesh,
      compiler_params=pltpu.CompilerParams(use_tc_tiling_on_sc=False),
      out_type=[
          jax.ShapeDtypeStruct((NUM_CORES, NPAD, IN_FEATS), jnp.float32),
          jax.ShapeDtypeStruct((NUM_CORES, NPAD, IN_FEATS), jnp.float32),
      ],
      scratch_types=[
          pltpu.VMEM((GROUP, IN_FEATS), jnp.float32),
          pltpu.VMEM((1, GROUP), jnp.int32),
          pltpu.VMEM((GROUP, IN_FEATS), jnp.float32),
          pltpu.VMEM((ZROWS, IN_FEATS), jnp.float32),
          pltpu.VMEM_SHARED((NPAD, IN_FEATS), jnp.float32),
          pltpu.VMEM_SHARED((NPAD, IN_FEATS), jnp.float32),
      ],
  )
  def sc_kernel(e_hbm, dst_hbm, seg_out, cnt_out,
                rows_v, idx_v, ones_v, zero_v, seg_sh, cnt_sh):
    c = lax.axis_index("c")
    s = lax.axis_index("s")
    wid = c * NUM_SUBCORES + s

    def init_body(i, _):
      zero_v[i] = jnp.zeros((IN_FEATS,), jnp.float32)
      return 0

    lax.fori_loop(0, ZROWS, init_body, 0)

    def ones_body(i, _):
      ones_v[i] = jnp.ones((IN_FEATS,), jnp.float32)
      return 0

    lax.fori_loop(0, GROUP, ones_body, 0)

    row0 = s * ROWS_PER_TILE
    for k in range(ROWS_PER_TILE // ZROWS):
      pltpu.sync_copy(zero_v, seg_sh.at[pl.ds(row0 + k * ZROWS, ZROWS)])
      pltpu.sync_copy(zero_v, cnt_sh.at[pl.ds(row0 + k * ZROWS, ZROWS)])
    rem = ROWS_PER_TILE % ZROWS
    if rem:
      rbase = row0 + (ROWS_PER_TILE // ZROWS) * ZROWS
      pltpu.sync_copy(zero_v.at[pl.ds(0, rem)], seg_sh.at[pl.ds(rbase, rem)])
      pltpu.sync_copy(zero_v.at[pl.ds(0, rem)], cnt_sh.at[pl.ds(rbase, rem)])
    plsc.subcore_barrier()

    g0 = wid * NUM_GROUPS // NUM_WORKERS
    g1 = (wid + 1) * NUM_GROUPS // NUM_WORKERS

    def group_body(g, _):
      base = g * GROUP
      pltpu.sync_copy(dst_hbm.at[pl.ds(base, GROUP)], idx_v.at[0])
      pltpu.sync_copy(e_hbm.at[pl.ds(base, GROUP)], rows_v)
      pltpu.sync_copy(rows_v, seg_sh.at[idx_v.at[0]], add=True)
      pltpu.sync_copy(ones_v, cnt_sh.at[idx_v.at[0]], add=True)
      return 0

    lax.fori_loop(g0, g1, group_body, 0)
    plsc.subcore_barrier()

    pltpu.sync_copy(seg_sh.at[pl.ds(row0, ROWS_PER_TILE)],
                    seg_out.at[c, pl.ds(row0, ROWS_PER_TILE)])
    pltpu.sync_copy(cnt_sh.at[pl.ds(row0, ROWS_PER_TILE)],
                    cnt_out.at[c, pl.ds(row0, ROWS_PER_TILE)])

  return sc_kernel(e, dst)


ROW_BLOCK = 2000


def _tc_body(seg_ref, cnt_ref, w_ref, b_ref, out_ref):
  seg = seg_ref[0] + seg_ref[1]
  cnt = cnt_ref[0, :, 0:1] + cnt_ref[1, :, 0:1]
  acc = jnp.dot(seg, w_ref[...], preferred_element_type=jnp.float32)
  acc = acc + cnt * b_ref[...]
  out_ref[...] = acc / jnp.maximum(cnt, 1.0)


def _tc_finish(seg, cnt, W, b2d):
  grid = N_NODES // ROW_BLOCK
  return pl.pallas_call(
      _tc_body,
      grid=(grid,),
      in_specs=[
          pl.BlockSpec((NUM_CORES, ROW_BLOCK, IN_FEATS), lambda i: (0, i, 0)),
          pl.BlockSpec((NUM_CORES, ROW_BLOCK, IN_FEATS), lambda i: (0, i, 0)),
          pl.BlockSpec((IN_FEATS, OUT_FEATS), lambda i: (0, 0)),
          pl.BlockSpec((1, OUT_FEATS), lambda i: (0, 0)),
      ],
      out_specs=pl.BlockSpec((ROW_BLOCK, OUT_FEATS), lambda i: (i, 0)),
      out_shape=jax.ShapeDtypeStruct((N_NODES, OUT_FEATS), jnp.float32),
  )(seg, cnt, W, b2d)


@jax.jit
def kernel(e, edge_index, W, b):
  dst = edge_index[1]
  seg, cnt = _sc_segment_sum(e, dst)
  return _tc_finish(seg, cnt, W, b.reshape(1, OUT_FEATS))


# speedup vs baseline: 8.8771x; 2.0408x over previous
import functools

import jax
import jax.numpy as jnp
from jax import lax
from jax.experimental import pallas as pl
from jax.experimental.pallas import tpu as pltpu
from jax.experimental.pallas import tpu_sc as plsc

N_NODES = 50000
N_EDGES = 800000
IN_FEATS = 16
OUT_FEATS = 100

NUM_CORES = 2
NUM_SUBCORES = 16
NUM_WORKERS = NUM_CORES * NUM_SUBCORES
GROUP = 128
NUM_GROUPS = N_EDGES // GROUP
NPAD = 51200
ROWS_PER_TILE = NPAD // NUM_SUBCORES
ZROWS = 400


def _sc_segment_sum(e3, dst):
  mesh = plsc.VectorSubcoreMesh(core_axis_name="c", subcore_axis_name="s")

  @functools.partial(
      pl.kernel,
      mesh=mesh,
      compiler_params=pltpu.CompilerParams(
          use_tc_tiling_on_sc=False, needs_layout_passes=False),
      out_type=[
          jax.ShapeDtypeStruct((NUM_CORES, NPAD, IN_FEATS), jnp.float32),
          jax.ShapeDtypeStruct((NUM_WORKERS, NPAD), jnp.float32),
      ],
      scratch_types=[
          pltpu.VMEM((2, IN_FEATS, GROUP), jnp.float32),
          pltpu.VMEM((2, 1, GROUP), jnp.int32),
          pltpu.VMEM((GROUP, IN_FEATS), jnp.float32),
          pltpu.VMEM((ZROWS, IN_FEATS), jnp.float32),
          pltpu.VMEM((NPAD,), jnp.float32),
          pltpu.VMEM_SHARED((NPAD, IN_FEATS), jnp.float32),
          pltpu.SemaphoreType.DMA((2,)),
      ],
  )
  def sc_kernel(e_hbm, dst_hbm, seg_out, cnt_out,
                a_v, idx_v, rows_v, zero_v, hist_v, seg_sh, lsem):
    c = lax.axis_index("c")
    s = lax.axis_index("s")
    wid = c * NUM_SUBCORES + s

    def zrow_body(i, _):
      zero_v[i] = jnp.zeros((IN_FEATS,), jnp.float32)
      return 0

    lax.fori_loop(0, ZROWS, zrow_body, 0)

    def hist_body(i, _):
      hist_v[pl.ds(i * IN_FEATS, IN_FEATS)] = jnp.zeros((IN_FEATS,),
                                                        jnp.float32)
      return 0

    lax.fori_loop(0, NPAD // IN_FEATS, hist_body, 0)

    row0 = s * ROWS_PER_TILE
    for k in range(ROWS_PER_TILE // ZROWS):
      pltpu.sync_copy(zero_v, seg_sh.at[pl.ds(row0 + k * ZROWS, ZROWS)])
    plsc.subcore_barrier()

    g0 = wid * NUM_GROUPS // NUM_WORKERS
    g1 = (wid + 1) * NUM_GROUPS // NUM_WORKERS

    def start_loads(g, b):
      pltpu.async_copy(e_hbm.at[g], a_v.at[b], lsem.at[b])
      pltpu.async_copy(dst_hbm.at[pl.ds(g * GROUP, GROUP)], idx_v.at[b, 0],
                       lsem.at[b])

    def wait_loads(g, b):
      pltpu.make_async_copy(e_hbm.at[g], a_v.at[b], lsem.at[b]).wait()
      pltpu.make_async_copy(dst_hbm.at[pl.ds(g * GROUP, GROUP)],
                            idx_v.at[b, 0], lsem.at[b]).wait()

    start_loads(g0, 0)
    iota16 = lax.iota(jnp.int32, 16)
    ones16 = jnp.ones((16,), jnp.float32)

    def group_body(g, _):
      b = (g - g0) % 2
      wait_loads(g, b)

      @pl.when(g + 1 < g1)
      def _():
        start_loads(g + 1, 1 - b)

      for f in range(IN_FEATS):
        for k in range(GROUP // 16):
          v = a_v[b, f, pl.ds(k * 16, 16)]
          plsc.store_scatter(
              rows_v, [iota16 + (k * 16), jnp.full((16,), f, jnp.int32)], v)
      for k in range(GROUP // 16):
        iv = idx_v[b, 0, pl.ds(k * 16, 16)]
        plsc.addupdate_scatter(hist_v, [iv], ones16)
      pltpu.sync_copy(rows_v, seg_sh.at[idx_v.at[b, 0]], add=True)
      return 0

    lax.fori_loop(g0, g1, group_body, 0)
    plsc.subcore_barrier()

    pltpu.sync_copy(seg_sh.at[pl.ds(row0, ROWS_PER_TILE)],
                    seg_out.at[c, pl.ds(row0, ROWS_PER_TILE)])
    pltpu.sync_copy(hist_v, cnt_out.at[wid])

  return sc_kernel(e3, dst)


ROW_BLOCK = 2048


def _tc_body(seg_ref, cnt_ref, w_ref, b_ref, out_ref):
  seg = seg_ref[0] + seg_ref[1]
  cnt = jnp.sum(jnp.transpose(cnt_ref[...]), axis=1, keepdims=True)
  acc = jnp.dot(seg, w_ref[...], preferred_element_type=jnp.float32)
  acc = acc + cnt * b_ref[...]
  out_ref[...] = acc / jnp.maximum(cnt, 1.0)


def _tc_finish(seg, cnt, W, b2d):
  grid = N_NODES // ROW_BLOCK + 1
  return pl.pallas_call(
      _tc_body,
      grid=(grid,),
      in_specs=[
          pl.BlockSpec((NUM_CORES, ROW_BLOCK, IN_FEATS), lambda i: (0, i, 0)),
          pl.BlockSpec((NUM_WORKERS, ROW_BLOCK), lambda i: (0, i)),
          pl.BlockSpec((IN_FEATS, OUT_FEATS), lambda i: (0, 0)),
          pl.BlockSpec((1, OUT_FEATS), lambda i: (0, 0)),
      ],
      out_specs=pl.BlockSpec((ROW_BLOCK, OUT_FEATS), lambda i: (i, 0)),
      out_shape=jax.ShapeDtypeStruct((N_NODES, OUT_FEATS), jnp.float32),
  )(seg, cnt, W, b2d)


@jax.jit
def kernel(e, edge_index, W, b):
  dst = edge_index[1]
  e3 = jnp.transpose(e.T.reshape(IN_FEATS, NUM_GROUPS, GROUP), (1, 0, 2))
  seg, cnt = _sc_segment_sum(e3, dst)
  return _tc_finish(seg, cnt, W, b.reshape(1, OUT_FEATS))


# speedup vs baseline: 10.5628x; 1.1899x over previous
import functools

import jax
import jax.numpy as jnp
from jax import lax
from jax.experimental import pallas as pl
from jax.experimental.pallas import tpu as pltpu
from jax.experimental.pallas import tpu_sc as plsc

N_NODES = 50000
N_EDGES = 800000
IN_FEATS = 16
OUT_FEATS = 100

NUM_CORES = 2
NUM_SUBCORES = 16
NUM_WORKERS = NUM_CORES * NUM_SUBCORES
GROUP = 128
NUM_GROUPS = N_EDGES // GROUP
NPAD = 51200
ROWS_PER_TILE = NPAD // NUM_SUBCORES
ZROWS = 400


def _sc_segment_sum(e3, dst):
  mesh = plsc.VectorSubcoreMesh(core_axis_name="c", subcore_axis_name="s")

  @functools.partial(
      pl.kernel,
      mesh=mesh,
      compiler_params=pltpu.CompilerParams(
          use_tc_tiling_on_sc=False, needs_layout_passes=False),
      out_type=[
          jax.ShapeDtypeStruct((NUM_CORES, NPAD, IN_FEATS), jnp.float32),
          jax.ShapeDtypeStruct((NUM_WORKERS, NPAD), jnp.float32),
      ],
      scratch_types=[
          pltpu.VMEM((2, 2, 8, GROUP), jnp.float32),
          pltpu.VMEM((2, 1, GROUP), jnp.int32),
          pltpu.VMEM((GROUP, IN_FEATS), jnp.float32),
          pltpu.VMEM((ZROWS, IN_FEATS), jnp.float32),
          pltpu.VMEM((NPAD,), jnp.float32),
          pltpu.VMEM_SHARED((NPAD, IN_FEATS), jnp.float32),
          pltpu.SemaphoreType.DMA((2,)),
      ],
  )
  def sc_kernel(e_hbm, dst_hbm, seg_out, cnt_out,
                a_v, idx_v, rows_v, zero_v, hist_v, seg_sh, lsem):
    c = lax.axis_index("c")
    s = lax.axis_index("s")
    wid = c * NUM_SUBCORES + s

    def zrow_body(i, _):
      zero_v[i] = jnp.zeros((IN_FEATS,), jnp.float32)
      return 0

    lax.fori_loop(0, ZROWS, zrow_body, 0)

    def hist_body(i, _):
      hist_v[pl.ds(i * IN_FEATS, IN_FEATS)] = jnp.zeros((IN_FEATS,),
                                                        jnp.float32)
      return 0

    lax.fori_loop(0, NPAD // IN_FEATS, hist_body, 0)

    row0 = s * ROWS_PER_TILE
    for k in range(ROWS_PER_TILE // ZROWS):
      pltpu.sync_copy(zero_v, seg_sh.at[pl.ds(row0 + k * ZROWS, ZROWS)])
    plsc.subcore_barrier()

    g0 = wid * NUM_GROUPS // NUM_WORKERS
    g1 = (wid + 1) * NUM_GROUPS // NUM_WORKERS

    def start_loads(g, b):
      pltpu.async_copy(e_hbm.at[0, g], a_v.at[b, 0], lsem.at[b])
      pltpu.async_copy(e_hbm.at[1, g], a_v.at[b, 1], lsem.at[b])
      pltpu.async_copy(dst_hbm.at[pl.ds(g * GROUP, GROUP)], idx_v.at[b, 0],
                       lsem.at[b])

    def wait_loads(g, b):
      pltpu.make_async_copy(e_hbm.at[0, g], a_v.at[b, 0], lsem.at[b]).wait()
      pltpu.make_async_copy(e_hbm.at[1, g], a_v.at[b, 1], lsem.at[b]).wait()
      pltpu.make_async_copy(dst_hbm.at[pl.ds(g * GROUP, GROUP)],
                            idx_v.at[b, 0], lsem.at[b]).wait()

    start_loads(g0, 0)
    iota16 = lax.iota(jnp.int32, 16)
    ones16 = jnp.ones((16,), jnp.float32)

    def group_body(g, _):
      b = (g - g0) % 2
      wait_loads(g, b)

      @pl.when(g + 1 < g1)
      def _():
        start_loads(g + 1, 1 - b)

      for f in range(IN_FEATS):
        for k in range(GROUP // 16):
          v = a_v[b, f // 8, f % 8, pl.ds(k * 16, 16)]
          plsc.store_scatter(
              rows_v, [iota16 + (k * 16), jnp.full((16,), f, jnp.int32)], v)
      for k in range(GROUP // 16):
        iv = idx_v[b, 0, pl.ds(k * 16, 16)]
        plsc.addupdate_scatter(hist_v, [iv], ones16)
      pltpu.sync_copy(rows_v, seg_sh.at[idx_v.at[b, 0]], add=True)
      return 0

    lax.fori_loop(g0, g1, group_body, 0)
    plsc.subcore_barrier()

    pltpu.sync_copy(seg_sh.at[pl.ds(row0, ROWS_PER_TILE)],
                    seg_out.at[c, pl.ds(row0, ROWS_PER_TILE)])
    pltpu.sync_copy(hist_v, cnt_out.at[wid])

  return sc_kernel(e3, dst)


ROW_BLOCK = 2048


def _tc_body(seg_ref, cnt_ref, wt_ref, b_ref, out_ref):
  seg = seg_ref[0] + seg_ref[1]
  seg_t = jnp.transpose(seg)
  cnt = jnp.sum(cnt_ref[...], axis=0, keepdims=True)
  acc = jnp.dot(wt_ref[...], seg_t, preferred_element_type=jnp.float32)
  acc = acc + b_ref[...] * cnt
  out_ref[...] = acc / jnp.maximum(cnt, 1.0)


def _tc_finish(seg, cnt, Wt, bcol):
  grid = N_NODES // ROW_BLOCK + 1
  return pl.pallas_call(
      _tc_body,
      grid=(grid,),
      in_specs=[
          pl.BlockSpec((NUM_CORES, ROW_BLOCK, IN_FEATS), lambda i: (0, i, 0)),
          pl.BlockSpec((NUM_WORKERS, ROW_BLOCK), lambda i: (0, i)),
          pl.BlockSpec((OUT_FEATS, IN_FEATS), lambda i: (0, 0)),
          pl.BlockSpec((OUT_FEATS, 1), lambda i: (0, 0)),
      ],
      out_specs=pl.BlockSpec((OUT_FEATS, ROW_BLOCK), lambda i: (0, i)),
      out_shape=jax.ShapeDtypeStruct((OUT_FEATS, N_NODES), jnp.float32),
  )(seg, cnt, Wt, bcol)


@jax.jit
def kernel(e, edge_index, W, b):
  dst = edge_index[1]
  e4 = jnp.transpose(e.T.reshape(2, 8, NUM_GROUPS, GROUP), (0, 2, 1, 3))
  seg, cnt = _sc_segment_sum(e4, dst)
  h_t = _tc_finish(seg, cnt, W.T, b.reshape(OUT_FEATS, 1))
  return h_t.T


# speedup vs baseline: 11.5271x; 1.0913x over previous
import functools

import jax
import jax.numpy as jnp
from jax import lax
from jax.experimental import pallas as pl
from jax.experimental.pallas import tpu as pltpu
from jax.experimental.pallas import tpu_sc as plsc

N_NODES = 50000
N_EDGES = 800000
IN_FEATS = 16
OUT_FEATS = 100

NUM_CORES = 2
NUM_SUBCORES = 16
NUM_WORKERS = NUM_CORES * NUM_SUBCORES
GROUP = 128
NUM_GROUPS = N_EDGES // GROUP
NPAD = 51200
ROWS_PER_TILE = NPAD // NUM_SUBCORES
ZROWS = 400


def _sc_segment_sum(e3, dst):
  mesh = plsc.VectorSubcoreMesh(core_axis_name="c", subcore_axis_name="s")

  @functools.partial(
      pl.kernel,
      mesh=mesh,
      compiler_params=pltpu.CompilerParams(
          use_tc_tiling_on_sc=False, needs_layout_passes=False),
      out_type=[
          jax.ShapeDtypeStruct((NUM_CORES, NPAD, IN_FEATS), jnp.float32),
          jax.ShapeDtypeStruct((NUM_WORKERS, NPAD), jnp.float32),
      ],
      scratch_types=[
          pltpu.VMEM((2, 2, 8, GROUP), jnp.float32),
          pltpu.VMEM((2, 1, GROUP), jnp.int32),
          pltpu.VMEM((GROUP, IN_FEATS), jnp.float32),
          pltpu.VMEM((ZROWS, IN_FEATS), jnp.float32),
          pltpu.VMEM((NPAD,), jnp.float32),
          pltpu.VMEM_SHARED((NPAD, IN_FEATS), jnp.float32),
          pltpu.SemaphoreType.DMA((2,)),
      ],
  )
  def sc_kernel(e_hbm, ei_hbm, seg_out, cnt_out,
                a_v, idx_v, rows_v, zero_v, hist_v, seg_sh, lsem):
    c = lax.axis_index("c")
    s = lax.axis_index("s")
    wid = c * NUM_SUBCORES + s

    def zrow_body(i, _):
      zero_v[i] = jnp.zeros((IN_FEATS,), jnp.float32)
      return 0

    lax.fori_loop(0, ZROWS, zrow_body, 0)

    def hist_body(i, _):
      hist_v[pl.ds(i * IN_FEATS, IN_FEATS)] = jnp.zeros((IN_FEATS,),
                                                        jnp.float32)
      return 0

    lax.fori_loop(0, NPAD // IN_FEATS, hist_body, 0)

    row0 = s * ROWS_PER_TILE
    for k in range(ROWS_PER_TILE // ZROWS):
      pltpu.sync_copy(zero_v, seg_sh.at[pl.ds(row0 + k * ZROWS, ZROWS)])
    plsc.subcore_barrier()

    g0 = wid * NUM_GROUPS // NUM_WORKERS
    g1 = (wid + 1) * NUM_GROUPS // NUM_WORKERS

    def start_loads(g, b):
      pltpu.async_copy(e_hbm.at[0, g], a_v.at[b, 0], lsem.at[b])
      pltpu.async_copy(e_hbm.at[1, g], a_v.at[b, 1], lsem.at[b])
      pltpu.async_copy(ei_hbm.at[g, 1], idx_v.at[b, 0], lsem.at[b])

    def wait_loads(g, b):
      pltpu.make_async_copy(e_hbm.at[0, g], a_v.at[b, 0], lsem.at[b]).wait()
      pltpu.make_async_copy(e_hbm.at[1, g], a_v.at[b, 1], lsem.at[b]).wait()
      pltpu.make_async_copy(ei_hbm.at[g, 1], idx_v.at[b, 0], lsem.at[b]).wait()

    start_loads(g0, 0)
    iota16 = lax.iota(jnp.int32, 16)
    ones16 = jnp.ones((16,), jnp.float32)

    def group_body(g, _):
      b = (g - g0) % 2
      wait_loads(g, b)

      @pl.when(g + 1 < g1)
      def _():
        start_loads(g + 1, 1 - b)

      for f in range(IN_FEATS):
        for k in range(GROUP // 16):
          v = a_v[b, f // 8, f % 8, pl.ds(k * 16, 16)]
          plsc.store_scatter(
              rows_v, [iota16 + (k * 16), jnp.full((16,), f, jnp.int32)], v)
      for k in range(GROUP // 16):
        iv = idx_v[b, 0, pl.ds(k * 16, 16)]
        plsc.addupdate_scatter(hist_v, [iv], ones16)
      pltpu.sync_copy(rows_v, seg_sh.at[idx_v.at[b, 0]], add=True)
      return 0

    lax.fori_loop(g0, g1, group_body, 0)
    plsc.subcore_barrier()

    pltpu.sync_copy(seg_sh.at[pl.ds(row0, ROWS_PER_TILE)],
                    seg_out.at[c, pl.ds(row0, ROWS_PER_TILE)])
    pltpu.sync_copy(hist_v, cnt_out.at[wid])

  return sc_kernel(e3, dst)


ROW_BLOCK = 2048


NSTEPS = N_NODES // ROW_BLOCK + 1


def _tc_body(seg_hbm, cnt_hbm, wt_ref, b_ref, out_ref, seg_v, cnt_v, sems):
  i = pl.program_id(0)

  def start(j, buf):
    pltpu.async_copy(seg_hbm.at[:, pl.ds(j * ROW_BLOCK, ROW_BLOCK), :],
                     seg_v.at[buf], sems.at[buf])
    pltpu.async_copy(cnt_hbm.at[:, pl.ds(j * ROW_BLOCK, ROW_BLOCK)],
                     cnt_v.at[buf], sems.at[buf])

  @pl.when(i == 0)
  def _():
    start(0, 0)

  @pl.when(i + 1 < NSTEPS)
  def _():
    start(i + 1, (i + 1) % 2)

  b = i % 2
  pltpu.make_async_copy(seg_hbm.at[:, pl.ds(i * ROW_BLOCK, ROW_BLOCK), :],
                        seg_v.at[b], sems.at[b]).wait()
  pltpu.make_async_copy(cnt_hbm.at[:, pl.ds(i * ROW_BLOCK, ROW_BLOCK)],
                        cnt_v.at[b], sems.at[b]).wait()
  seg = seg_v[b, 0] + seg_v[b, 1]
  seg_t = jnp.transpose(seg)
  cnt = jnp.sum(cnt_v[b], axis=0, keepdims=True)
  acc = jnp.dot(wt_ref[...], seg_t, preferred_element_type=jnp.float32)
  acc = acc + b_ref[...] * cnt
  out_ref[...] = acc / jnp.maximum(cnt, 1.0)


def _tc_finish(seg, cnt, Wt, bcol):
  return pl.pallas_call(
      _tc_body,
      grid=(NSTEPS,),
      in_specs=[
          pl.BlockSpec(memory_space=pl.ANY),
          pl.BlockSpec(memory_space=pl.ANY),
          pl.BlockSpec((OUT_FEATS, IN_FEATS), lambda i: (0, 0)),
          pl.BlockSpec((OUT_FEATS, 1), lambda i: (0, 0)),
      ],
      out_specs=pl.BlockSpec((OUT_FEATS, ROW_BLOCK), lambda i: (0, i)),
      out_shape=jax.ShapeDtypeStruct((OUT_FEATS, N_NODES), jnp.float32),
      scratch_shapes=[
          pltpu.VMEM((2, NUM_CORES, ROW_BLOCK, IN_FEATS), jnp.float32),
          pltpu.VMEM((2, NUM_WORKERS, ROW_BLOCK), jnp.float32),
          pltpu.SemaphoreType.DMA((2,)),
      ],
  )(seg, cnt, Wt, bcol)


@jax.jit
def kernel(e, edge_index, W, b):
  e4 = jnp.transpose(e.T.reshape(2, 8, NUM_GROUPS, GROUP), (0, 2, 1, 3))
  ei5 = jnp.transpose(edge_index.reshape(2, NUM_GROUPS, GROUP), (1, 0, 2))
  seg, cnt = _sc_segment_sum(e4, ei5)
  h_t = _tc_finish(seg, cnt, W.T, b.reshape(OUT_FEATS, 1))
  return h_t.T


# speedup vs baseline: 12.8355x; 1.1135x over previous
import functools

import jax
import jax.numpy as jnp
from jax import lax
from jax.experimental import pallas as pl
from jax.experimental.pallas import tpu as pltpu
from jax.experimental.pallas import tpu_sc as plsc

N_NODES = 50000
N_EDGES = 800000
IN_FEATS = 16
OUT_FEATS = 100

NUM_CORES = 2
NUM_SUBCORES = 16
NUM_WORKERS = NUM_CORES * NUM_SUBCORES
GROUP = 128
NUM_GROUPS = N_EDGES // GROUP
NPAD = 51200
ROWS_PER_TILE = NPAD // NUM_SUBCORES
ZROWS = 400


def _sc_segment_sum(e3, dst):
  mesh = plsc.VectorSubcoreMesh(core_axis_name="c", subcore_axis_name="s")

  @functools.partial(
      pl.kernel,
      mesh=mesh,
      compiler_params=pltpu.CompilerParams(
          use_tc_tiling_on_sc=False, needs_layout_passes=False),
      out_type=[
          jax.ShapeDtypeStruct((NUM_CORES, NPAD, IN_FEATS), jnp.float32),
          jax.ShapeDtypeStruct((NUM_WORKERS, NPAD), jnp.float32),
      ],
      scratch_types=[
          pltpu.VMEM((2, 2, 8, GROUP), jnp.float32),
          pltpu.VMEM((2, 1, GROUP), jnp.int32),
          pltpu.VMEM((GROUP, IN_FEATS), jnp.float32),
          pltpu.VMEM((ZROWS, IN_FEATS), jnp.float32),
          pltpu.VMEM((NPAD,), jnp.float32),
          pltpu.VMEM_SHARED((NPAD, IN_FEATS), jnp.float32),
          pltpu.SemaphoreType.DMA((2,)),
      ],
  )
  def sc_kernel(e_hbm, ei_hbm, seg_out, cnt_out,
                a_v, idx_v, rows_v, zero_v, hist_v, seg_sh, lsem):
    c = lax.axis_index("c")
    s = lax.axis_index("s")
    wid = c * NUM_SUBCORES + s

    def zrow_body(i, _):
      zero_v[i] = jnp.zeros((IN_FEATS,), jnp.float32)
      return 0

    lax.fori_loop(0, ZROWS, zrow_body, 0)

    def hist_body(i, _):
      hist_v[pl.ds(i * IN_FEATS, IN_FEATS)] = jnp.zeros((IN_FEATS,),
                                                        jnp.float32)
      return 0

    lax.fori_loop(0, NPAD // IN_FEATS, hist_body, 0)

    row0 = s * ROWS_PER_TILE
    for k in range(ROWS_PER_TILE // ZROWS):
      pltpu.sync_copy(zero_v, seg_sh.at[pl.ds(row0 + k * ZROWS, ZROWS)])
    plsc.subcore_barrier()

    g0 = wid * NUM_GROUPS // NUM_WORKERS
    g1 = (wid + 1) * NUM_GROUPS // NUM_WORKERS

    def start_loads(g, b):
      pltpu.async_copy(e_hbm.at[0, g], a_v.at[b, 0], lsem.at[b])
      pltpu.async_copy(e_hbm.at[1, g], a_v.at[b, 1], lsem.at[b])
      pltpu.async_copy(ei_hbm.at[g, 1], idx_v.at[b, 0], lsem.at[b])

    def wait_loads(g, b):
      pltpu.make_async_copy(e_hbm.at[0, g], a_v.at[b, 0], lsem.at[b]).wait()
      pltpu.make_async_copy(e_hbm.at[1, g], a_v.at[b, 1], lsem.at[b]).wait()
      pltpu.make_async_copy(ei_hbm.at[g, 1], idx_v.at[b, 0], lsem.at[b]).wait()

    start_loads(g0, 0)
    iota16 = lax.iota(jnp.int32, 16)
    ones16 = jnp.ones((16,), jnp.float32)

    def group_body(g, _):
      b = (g - g0) % 2
      wait_loads(g, b)

      @pl.when(g + 1 < g1)
      def _():
        start_loads(g + 1, 1 - b)

      for f in range(IN_FEATS):
        for k in range(GROUP // 16):
          v = a_v[b, f // 8, f % 8, pl.ds(k * 16, 16)]
          plsc.store_scatter(
              rows_v, [iota16 + (k * 16), jnp.full((16,), f, jnp.int32)], v)
      for k in range(GROUP // 16):
        iv = idx_v[b, 0, pl.ds(k * 16, 16)]
        plsc.addupdate_scatter(hist_v, [iv], ones16)
      pltpu.sync_copy(rows_v, seg_sh.at[idx_v.at[b, 0]], add=True)
      return 0

    lax.fori_loop(g0, g1, group_body, 0)
    plsc.subcore_barrier()

    pltpu.sync_copy(seg_sh.at[pl.ds(row0, ROWS_PER_TILE)],
                    seg_out.at[c, pl.ds(row0, ROWS_PER_TILE)])
    pltpu.sync_copy(hist_v, cnt_out.at[wid])

  return sc_kernel(e3, dst)


ROW_BLOCK = 2048


NSTEPS = N_NODES // ROW_BLOCK + 1
PACK = ROW_BLOCK // 8


def _tc_body(seg_ref, cnt_ref, wt_ref, b_ref, out_ref):
  p = seg_ref[0] + seg_ref[1]
  parts = [p[:, 16 * a:16 * a + 16] for a in range(8)]
  seg = jnp.stack(parts, axis=1).reshape(ROW_BLOCK, IN_FEATS)
  seg_t = jnp.transpose(seg)
  cnt = jnp.sum(cnt_ref[...], axis=0, keepdims=True)
  acc = jnp.dot(wt_ref[...], seg_t, preferred_element_type=jnp.float32)
  acc = acc + b_ref[...] * cnt
  out_ref[...] = acc / jnp.maximum(cnt, 1.0)


def _tc_finish(seg2, cnt, Wt, bcol):
  return pl.pallas_call(
      _tc_body,
      grid=(NSTEPS,),
      in_specs=[
          pl.BlockSpec((NUM_CORES, PACK, GROUP), lambda i: (0, i, 0)),
          pl.BlockSpec((NUM_WORKERS, ROW_BLOCK), lambda i: (0, i)),
          pl.BlockSpec((OUT_FEATS, IN_FEATS), lambda i: (0, 0)),
          pl.BlockSpec((OUT_FEATS, 1), lambda i: (0, 0)),
      ],
      out_specs=pl.BlockSpec((OUT_FEATS, ROW_BLOCK), lambda i: (0, i)),
      out_shape=jax.ShapeDtypeStruct((OUT_FEATS, N_NODES), jnp.float32),
  )(seg2, cnt, Wt, bcol)


@jax.jit
def kernel(e, edge_index, W, b):
  e4 = jnp.transpose(e.T.reshape(2, 8, NUM_GROUPS, GROUP), (0, 2, 1, 3))
  ei5 = jnp.transpose(edge_index.reshape(2, NUM_GROUPS, GROUP), (1, 0, 2))
  seg, cnt = _sc_segment_sum(e4, ei5)
  seg2 = seg.reshape(NUM_CORES, NPAD // 8, GROUP)
  h_t = _tc_finish(seg2, cnt, W.T, b.reshape(OUT_FEATS, 1))
  return h_t.T


# speedup vs baseline: 13.7560x; 1.0717x over previous
import functools

import jax
import jax.numpy as jnp
from jax import lax
from jax.experimental import pallas as pl
from jax.experimental.pallas import tpu as pltpu
from jax.experimental.pallas import tpu_sc as plsc

N_NODES = 50000
N_EDGES = 800000
IN_FEATS = 16
OUT_FEATS = 100

NUM_CORES = 2
NUM_SUBCORES = 16
NUM_WORKERS = NUM_CORES * NUM_SUBCORES
GROUP = 128
NUM_GROUPS = N_EDGES // GROUP
NPAD = 51200
ROWS_PER_TILE = NPAD // NUM_SUBCORES
ZROWS = 400


def _sc_segment_sum(e3, dst):
  mesh = plsc.VectorSubcoreMesh(core_axis_name="c", subcore_axis_name="s")

  @functools.partial(
      pl.kernel,
      mesh=mesh,
      compiler_params=pltpu.CompilerParams(
          use_tc_tiling_on_sc=False, needs_layout_passes=False),
      out_type=[
          jax.ShapeDtypeStruct((NUM_CORES, NPAD, IN_FEATS), jnp.float32),
          jax.ShapeDtypeStruct((NUM_WORKERS, NPAD), jnp.float32),
      ],
      scratch_types=[
          pltpu.VMEM((2, 2, 8, GROUP), jnp.float32),
          pltpu.VMEM((3, 1, GROUP), jnp.int32),
          pltpu.VMEM((2, GROUP, IN_FEATS), jnp.float32),
          pltpu.VMEM((ZROWS, IN_FEATS), jnp.float32),
          pltpu.VMEM((NPAD,), jnp.float32),
          pltpu.VMEM_SHARED((NPAD, IN_FEATS), jnp.float32),
          pltpu.SemaphoreType.DMA((2,)),
          pltpu.SemaphoreType.DMA((2,)),
      ],
  )
  def sc_kernel(e_hbm, ei_hbm, seg_out, cnt_out,
                a_v, idx_v, rows_v, zero_v, hist_v, seg_sh, lsem, ssem):
    c = lax.axis_index("c")
    s = lax.axis_index("s")
    wid = c * NUM_SUBCORES + s

    def zrow_body(i, _):
      zero_v[i] = jnp.zeros((IN_FEATS,), jnp.float32)
      return 0

    lax.fori_loop(0, ZROWS, zrow_body, 0)

    def hist_body(i, _):
      hist_v[pl.ds(i * IN_FEATS, IN_FEATS)] = jnp.zeros((IN_FEATS,),
                                                        jnp.float32)
      return 0

    lax.fori_loop(0, NPAD // IN_FEATS, hist_body, 0)

    row0 = s * ROWS_PER_TILE
    for k in range(ROWS_PER_TILE // ZROWS):
      pltpu.sync_copy(zero_v, seg_sh.at[pl.ds(row0 + k * ZROWS, ZROWS)])
    plsc.subcore_barrier()

    g0 = wid * NUM_GROUPS // NUM_WORKERS
    g1 = (wid + 1) * NUM_GROUPS // NUM_WORKERS

    def start_loads(g, b2, b3):
      pltpu.async_copy(e_hbm.at[0, g], a_v.at[b2, 0], lsem.at[b2])
      pltpu.async_copy(e_hbm.at[1, g], a_v.at[b2, 1], lsem.at[b2])
      pltpu.async_copy(ei_hbm.at[g, 1], idx_v.at[b3, 0], lsem.at[b2])

    def wait_loads(g, b2, b3):
      pltpu.make_async_copy(e_hbm.at[0, g], a_v.at[b2, 0], lsem.at[b2]).wait()
      pltpu.make_async_copy(e_hbm.at[1, g], a_v.at[b2, 1], lsem.at[b2]).wait()
      pltpu.make_async_copy(ei_hbm.at[g, 1], idx_v.at[b3, 0],
                            lsem.at[b2]).wait()

    def wait_scatter(g):
      b2 = (g - g0) % 2
      b3 = (g - g0) % 3
      pltpu.make_async_copy(rows_v.at[b2], seg_sh.at[idx_v.at[b3, 0]],
                            ssem.at[b2]).wait()

    start_loads(g0, 0, 0)
    iota16 = lax.iota(jnp.int32, 16)
    ones16 = jnp.ones((16,), jnp.float32)

    def group_body(g, _):
      b2 = (g - g0) % 2
      b3 = (g - g0) % 3
      wait_loads(g, b2, b3)

      @pl.when(g + 1 < g1)
      def _():
        start_loads(g + 1, 1 - b2, (g + 1 - g0) % 3)

      for f in range(IN_FEATS):
        for k in range(GROUP // 16):
          v = a_v[b2, f // 8, f % 8, pl.ds(k * 16, 16)]
          plsc.store_scatter(
              rows_v.at[b2],
              [iota16 + (k * 16), jnp.full((16,), f, jnp.int32)], v)
      for k in range(GROUP // 16):
        iv = idx_v[b3, 0, pl.ds(k * 16, 16)]
        plsc.addupdate_scatter(hist_v, [iv], ones16)

      @pl.when(g > g0)
      def _():
        wait_scatter(g - 1)

      pltpu.async_copy(rows_v.at[b2], seg_sh.at[idx_v.at[b3, 0]],
                       ssem.at[b2], add=True)
      return 0

    lax.fori_loop(g0, g1, group_body, 0)
    wait_scatter(g1 - 1)
    plsc.subcore_barrier()

    pltpu.sync_copy(seg_sh.at[pl.ds(row0, ROWS_PER_TILE)],
                    seg_out.at[c, pl.ds(row0, ROWS_PER_TILE)])
    pltpu.sync_copy(hist_v, cnt_out.at[wid])

  return sc_kernel(e3, dst)


ROW_BLOCK = 2048


NSTEPS = N_NODES // ROW_BLOCK + 1
PACK = ROW_BLOCK // 8


def _tc_body(seg_ref, cnt_ref, wt_ref, b_ref, out_ref):
  p = seg_ref[0] + seg_ref[1]
  parts = [p[:, 16 * a:16 * a + 16] for a in range(8)]
  seg = jnp.stack(parts, axis=1).reshape(ROW_BLOCK, IN_FEATS)
  seg_t = jnp.transpose(seg)
  cnt = jnp.sum(cnt_ref[...], axis=0, keepdims=True)
  acc = jnp.dot(wt_ref[...], seg_t, preferred_element_type=jnp.float32)
  acc = acc + b_ref[...] * cnt
  out_ref[...] = acc / jnp.maximum(cnt, 1.0)


def _tc_finish(seg2, cnt, Wt, bcol):
  return pl.pallas_call(
      _tc_body,
      grid=(NSTEPS,),
      in_specs=[
          pl.BlockSpec((NUM_CORES, PACK, GROUP), lambda i: (0, i, 0)),
          pl.BlockSpec((NUM_WORKERS, ROW_BLOCK), lambda i: (0, i)),
          pl.BlockSpec((OUT_FEATS, IN_FEATS), lambda i: (0, 0)),
          pl.BlockSpec((OUT_FEATS, 1), lambda i: (0, 0)),
      ],
      out_specs=pl.BlockSpec((OUT_FEATS, ROW_BLOCK), lambda i: (0, i)),
      out_shape=jax.ShapeDtypeStruct((OUT_FEATS, N_NODES), jnp.float32),
  )(seg2, cnt, Wt, bcol)


@jax.jit
def kernel(e, edge_index, W, b):
  e4 = jnp.transpose(e.T.reshape(2, 8, NUM_GROUPS, GROUP), (0, 2, 1, 3))
  ei5 = jnp.transpose(edge_index.reshape(2, NUM_GROUPS, GROUP), (1, 0, 2))
  seg, cnt = _sc_segment_sum(e4, ei5)
  seg2 = seg.reshape(NUM_CORES, NPAD // 8, GROUP)
  h_t = _tc_finish(seg2, cnt, W.T, b.reshape(OUT_FEATS, 1))
  return h_t.T


# speedup vs baseline: 13.7581x; 1.0002x over previous
import functools

import jax
import jax.numpy as jnp
from jax import lax
from jax.experimental import pallas as pl
from jax.experimental.pallas import tpu as pltpu
from jax.experimental.pallas import tpu_sc as plsc

N_NODES = 50000
N_EDGES = 800000
IN_FEATS = 16
OUT_FEATS = 100

NUM_CORES = 2
NUM_SUBCORES = 16
NUM_WORKERS = NUM_CORES * NUM_SUBCORES
GROUP = 128
NUM_GROUPS = N_EDGES // GROUP
NPAD = 51200
ROWS_PER_TILE = NPAD // NUM_SUBCORES
ZROWS = 400


def _sc_segment_sum(e3, dst):
  mesh = plsc.VectorSubcoreMesh(core_axis_name="c", subcore_axis_name="s")

  @functools.partial(
      pl.kernel,
      mesh=mesh,
      compiler_params=pltpu.CompilerParams(
          use_tc_tiling_on_sc=False, needs_layout_passes=False),
      out_type=[
          jax.ShapeDtypeStruct((NUM_CORES, NPAD, IN_FEATS), jnp.float32),
          jax.ShapeDtypeStruct((NUM_WORKERS, NPAD), jnp.float32),
      ],
      scratch_types=[
          pltpu.VMEM((2, 2, 8, GROUP), jnp.float32),
          pltpu.VMEM((3, 1, GROUP), jnp.int32),
          pltpu.VMEM((2, GROUP, IN_FEATS), jnp.float32),
          pltpu.VMEM((ZROWS, IN_FEATS), jnp.float32),
          pltpu.VMEM((NPAD,), jnp.float32),
          pltpu.VMEM_SHARED((NPAD, IN_FEATS), jnp.float32),
          pltpu.SemaphoreType.DMA((2,)),
          pltpu.SemaphoreType.DMA((2,)),
      ],
  )
  def sc_kernel(e_hbm, ei_hbm, seg_out, cnt_out,
                a_v, idx_v, rows_v, zero_v, hist_v, seg_sh, lsem, ssem):
    c = lax.axis_index("c")
    s = lax.axis_index("s")
    wid = c * NUM_SUBCORES + s

    def zrow_body(i, _):
      zero_v[i] = jnp.zeros((IN_FEATS,), jnp.float32)
      return 0

    lax.fori_loop(0, ZROWS, zrow_body, 0)

    def hist_body(i, _):
      hist_v[pl.ds(i * IN_FEATS, IN_FEATS)] = jnp.zeros((IN_FEATS,),
                                                        jnp.float32)
      return 0

    lax.fori_loop(0, NPAD // IN_FEATS, hist_body, 0)

    row0 = s * ROWS_PER_TILE
    for k in range(ROWS_PER_TILE // ZROWS):
      pltpu.sync_copy(zero_v, seg_sh.at[pl.ds(row0 + k * ZROWS, ZROWS)])
    plsc.subcore_barrier()

    g0 = wid * NUM_GROUPS // NUM_WORKERS
    g1 = (wid + 1) * NUM_GROUPS // NUM_WORKERS

    def start_loads(g, b2, b3):
      pltpu.async_copy(e_hbm.at[0, g], a_v.at[b2, 0], lsem.at[b2])
      pltpu.async_copy(e_hbm.at[1, g], a_v.at[b2, 1], lsem.at[b2])
      pltpu.async_copy(ei_hbm.at[g, 1], idx_v.at[b3, 0], lsem.at[b2])

    def wait_loads(g, b2, b3):
      pltpu.make_async_copy(e_hbm.at[0, g], a_v.at[b2, 0], lsem.at[b2]).wait()
      pltpu.make_async_copy(e_hbm.at[1, g], a_v.at[b2, 1], lsem.at[b2]).wait()
      pltpu.make_async_copy(ei_hbm.at[g, 1], idx_v.at[b3, 0],
                            lsem.at[b2]).wait()

    def wait_scatter(g):
      b2 = (g - g0) % 2
      b3 = (g - g0) % 3
      pltpu.make_async_copy(rows_v.at[b2], seg_sh.at[idx_v.at[b3, 0]],
                            ssem.at[b2]).wait()

    start_loads(g0, 0, 0)
    iota16 = lax.iota(jnp.int32, 16)
    ones16 = jnp.ones((16,), jnp.float32)

    def group_body(g, _):
      b2 = (g - g0) % 2
      b3 = (g - g0) % 3
      wait_loads(g, b2, b3)

      @pl.when(g + 1 < g1)
      def _():
        start_loads(g + 1, 1 - b2, (g + 1 - g0) % 3)

      for f in range(IN_FEATS):
        for k in range(GROUP // 16):
          v = a_v[b2, f // 8, f % 8, pl.ds(k * 16, 16)]
          plsc.store_scatter(
              rows_v.at[b2],
              [iota16 + (k * 16), jnp.full((16,), f, jnp.int32)], v)
      for k in range(GROUP // 16):
        iv = idx_v[b3, 0, pl.ds(k * 16, 16)]
        plsc.addupdate_scatter(hist_v, [iv], ones16)

      @pl.when(g > g0)
      def _():
        wait_scatter(g - 1)

      pltpu.async_copy(rows_v.at[b2], seg_sh.at[idx_v.at[b3, 0]],
                       ssem.at[b2], add=True)
      return 0

    lax.fori_loop(g0, g1, group_body, 0)
    wait_scatter(g1 - 1)
    pltpu.sync_copy(hist_v, cnt_out.at[wid])
    plsc.subcore_barrier()

    pltpu.sync_copy(seg_sh.at[pl.ds(row0, ROWS_PER_TILE)],
                    seg_out.at[c, pl.ds(row0, ROWS_PER_TILE)])

  return sc_kernel(e3, dst)


ROW_BLOCK = 2048


NSTEPS = N_NODES // ROW_BLOCK + 1
PACK = ROW_BLOCK // 8


def _tc_body(seg_ref, cnt_ref, wt_ref, b_ref, out_ref):
  p = seg_ref[0] + seg_ref[1]
  parts = [p[:, 16 * a:16 * a + 16] for a in range(8)]
  seg = jnp.stack(parts, axis=1).reshape(ROW_BLOCK, IN_FEATS)
  cnt = jnp.sum(cnt_ref[...], axis=0, keepdims=True)
  acc = lax.dot_general(wt_ref[...], seg,
                        (((1,), (1,)), ((), ())),
                        preferred_element_type=jnp.float32)
  acc = acc + b_ref[...] * cnt
  out_ref[...] = acc / jnp.maximum(cnt, 1.0)


def _tc_finish(seg2, cnt, Wt, bcol):
  return pl.pallas_call(
      _tc_body,
      grid=(NSTEPS,),
      in_specs=[
          pl.BlockSpec((NUM_CORES, PACK, GROUP), lambda i: (0, i, 0)),
          pl.BlockSpec((NUM_WORKERS, ROW_BLOCK), lambda i: (0, i)),
          pl.BlockSpec((OUT_FEATS, IN_FEATS), lambda i: (0, 0)),
          pl.BlockSpec((OUT_FEATS, 1), lambda i: (0, 0)),
      ],
      out_specs=pl.BlockSpec((OUT_FEATS, ROW_BLOCK), lambda i: (0, i)),
      out_shape=jax.ShapeDtypeStruct((OUT_FEATS, N_NODES), jnp.float32),
  )(seg2, cnt, Wt, bcol)


@jax.jit
def kernel(e, edge_index, W, b):
  e4 = jnp.transpose(e.T.reshape(2, 8, NUM_GROUPS, GROUP), (0, 2, 1, 3))
  ei5 = jnp.transpose(edge_index.reshape(2, NUM_GROUPS, GROUP), (1, 0, 2))
  seg, cnt = _sc_segment_sum(e4, ei5)
  seg2 = seg.reshape(NUM_CORES, NPAD // 8, GROUP)
  h_t = _tc_finish(seg2, cnt, W.T, b.reshape(OUT_FEATS, 1))
  return h_t.T


# speedup vs baseline: 13.8433x; 1.0062x over previous
import functools

import jax
import jax.numpy as jnp
from jax import lax
from jax.experimental import pallas as pl
from jax.experimental.pallas import tpu as pltpu
from jax.experimental.pallas import tpu_sc as plsc

N_NODES = 50000
N_EDGES = 800000
IN_FEATS = 16
OUT_FEATS = 100

NUM_CORES = 2
NUM_SUBCORES = 16
NUM_WORKERS = NUM_CORES * NUM_SUBCORES
GROUP = 128
NUM_GROUPS = N_EDGES // GROUP
NPAD = 51200
ROWS_PER_TILE = NPAD // NUM_SUBCORES
ZROWS = 400


def _sc_segment_sum(e3, dst):
  mesh = plsc.VectorSubcoreMesh(core_axis_name="c", subcore_axis_name="s")

  @functools.partial(
      pl.kernel,
      mesh=mesh,
      compiler_params=pltpu.CompilerParams(
          use_tc_tiling_on_sc=False, needs_layout_passes=False),
      out_type=[
          jax.ShapeDtypeStruct((NUM_CORES, NPAD, IN_FEATS), jnp.float32),
          jax.ShapeDtypeStruct((NUM_WORKERS, NPAD), jnp.float32),
      ],
      scratch_types=[
          pltpu.VMEM((2, 2, 8, GROUP), jnp.float32),
          pltpu.VMEM((3, 1, GROUP), jnp.int32),
          pltpu.VMEM((2, GROUP, IN_FEATS), jnp.float32),
          pltpu.VMEM((ZROWS, IN_FEATS), jnp.float32),
          pltpu.VMEM((NPAD,), jnp.float32),
          pltpu.VMEM_SHARED((NPAD, IN_FEATS), jnp.float32),
          pltpu.SemaphoreType.DMA((2,)),
          pltpu.SemaphoreType.DMA((2,)),
      ],
  )
  def sc_kernel(e_hbm, ei_hbm, seg_out, cnt_out,
                a_v, idx_v, rows_v, zero_v, hist_v, seg_sh, lsem, ssem):
    c = lax.axis_index("c")
    s = lax.axis_index("s")
    wid = c * NUM_SUBCORES + s

    def zrow_body(i, _):
      zero_v[i] = jnp.zeros((IN_FEATS,), jnp.float32)
      return 0

    lax.fori_loop(0, ZROWS, zrow_body, 0)

    def hist_body(i, _):
      hist_v[pl.ds(i * IN_FEATS, IN_FEATS)] = jnp.zeros((IN_FEATS,),
                                                        jnp.float32)
      return 0

    lax.fori_loop(0, NPAD // IN_FEATS, hist_body, 0)

    row0 = s * ROWS_PER_TILE
    for k in range(ROWS_PER_TILE // ZROWS):
      pltpu.sync_copy(zero_v, seg_sh.at[pl.ds(row0 + k * ZROWS, ZROWS)])
    plsc.subcore_barrier()

    g0 = wid * NUM_GROUPS // NUM_WORKERS
    g1 = (wid + 1) * NUM_GROUPS // NUM_WORKERS

    def start_loads(g, b2, b3):
      pltpu.async_copy(e_hbm.at[0, g], a_v.at[b2, 0], lsem.at[b2])
      pltpu.async_copy(e_hbm.at[1, g], a_v.at[b2, 1], lsem.at[b2])
      pltpu.async_copy(ei_hbm.at[g, 1], idx_v.at[b3, 0], lsem.at[b2])

    def wait_loads(g, b2, b3):
      pltpu.make_async_copy(e_hbm.at[0, g], a_v.at[b2, 0], lsem.at[b2]).wait()
      pltpu.make_async_copy(e_hbm.at[1, g], a_v.at[b2, 1], lsem.at[b2]).wait()
      pltpu.make_async_copy(ei_hbm.at[g, 1], idx_v.at[b3, 0],
                            lsem.at[b2]).wait()

    def wait_scatter(g):
      b2 = (g - g0) % 2
      b3 = (g - g0) % 3
      pltpu.make_async_copy(rows_v.at[b2], seg_sh.at[idx_v.at[b3, 0]],
                            ssem.at[b2]).wait()

    start_loads(g0, 0, 0)
    iota16 = lax.iota(jnp.int32, 16)
    ones16 = jnp.ones((16,), jnp.float32)

    def group_body(g, _):
      b2 = (g - g0) % 2
      b3 = (g - g0) % 3
      wait_loads(g, b2, b3)

      @pl.when(g + 1 < g1)
      def _():
        start_loads(g + 1, 1 - b2, (g + 1 - g0) % 3)

      for f in range(IN_FEATS):
        for k in range(GROUP // 16):
          v = a_v[b2, f // 8, f % 8, pl.ds(k * 16, 16)]
          plsc.store_scatter(
              rows_v.at[b2],
              [iota16 + (k * 16), jnp.full((16,), f, jnp.int32)], v)
      for k in range(GROUP // 16):
        iv = idx_v[b3, 0, pl.ds(k * 16, 16)]
        plsc.addupdate_scatter(hist_v, [iv], ones16)

      @pl.when(g > g0)
      def _():
        wait_scatter(g - 1)

      pltpu.async_copy(rows_v.at[b2], seg_sh.at[idx_v.at[b3, 0]],
                       ssem.at[b2], add=True)
      return 0

    lax.fori_loop(g0, g1, group_body, 0)
    wait_scatter(g1 - 1)
    pltpu.sync_copy(hist_v, cnt_out.at[wid])
    plsc.subcore_barrier()

    pltpu.sync_copy(seg_sh.at[pl.ds(row0, ROWS_PER_TILE)],
                    seg_out.at[c, pl.ds(row0, ROWS_PER_TILE)])

  return sc_kernel(e3, dst)


ROW_BLOCK = 3200


NSTEPS = NPAD // ROW_BLOCK
PACK = ROW_BLOCK // 8


def _tc_body(seg_ref, cnt_ref, wt_ref, b_ref, out_ref):
  p = seg_ref[0] + seg_ref[1]
  parts = [p[:, 16 * a:16 * a + 16] for a in range(8)]
  seg = jnp.stack(parts, axis=1).reshape(ROW_BLOCK, IN_FEATS)
  cnt = jnp.sum(cnt_ref[...], axis=0, keepdims=True)
  acc = lax.dot_general(wt_ref[...], seg,
                        (((1,), (1,)), ((), ())),
                        preferred_element_type=jnp.float32)
  acc = acc + b_ref[...] * cnt
  out_ref[...] = acc / jnp.maximum(cnt, 1.0)


def _tc_finish(seg2, cnt, Wt, bcol):
  return pl.pallas_call(
      _tc_body,
      grid=(NSTEPS,),
      in_specs=[
          pl.BlockSpec((NUM_CORES, PACK, GROUP), lambda i: (0, i, 0)),
          pl.BlockSpec((NUM_WORKERS, ROW_BLOCK), lambda i: (0, i)),
          pl.BlockSpec((OUT_FEATS, IN_FEATS), lambda i: (0, 0)),
          pl.BlockSpec((OUT_FEATS, 1), lambda i: (0, 0)),
      ],
      out_specs=pl.BlockSpec((OUT_FEATS, ROW_BLOCK), lambda i: (0, i)),
      out_shape=jax.ShapeDtypeStruct((OUT_FEATS, N_NODES), jnp.float32),
  )(seg2, cnt, Wt, bcol)


@jax.jit
def kernel(e, edge_index, W, b):
  e4 = jnp.transpose(e.T.reshape(2, 8, NUM_GROUPS, GROUP), (0, 2, 1, 3))
  ei5 = jnp.transpose(edge_index.reshape(2, NUM_GROUPS, GROUP), (1, 0, 2))
  seg, cnt = _sc_segment_sum(e4, ei5)
  seg2 = seg.reshape(NUM_CORES, NPAD // 8, GROUP)
  h_t = _tc_finish(seg2, cnt, W.T, b.reshape(OUT_FEATS, 1))
  return h_t.T
